# baseline (speedup 1.0000x reference)
import functools

import jax
import jax.numpy as jnp
from jax import lax
from jax.experimental import pallas as pl
from jax.experimental.pallas import tpu as pltpu
from jax.experimental.pallas import tpu_sc as plsc

_B, _ST, _SS, _P, _D = 2, 8, 4, 1369, 1024
_R = 128
_S = 128
_K = 4
_SHT = (2, 4, 6)
_SHS = (1, 2, 3)
_EPS = 1e-8
_RK = _R * _K
_PC = 344
_NC = 4
_NCAND = _NC * _K


def _sc_perm_gather(tt3, st4, perm_tab):
    info = plsc.get_sparse_core_info()
    nw = info.num_cores * info.num_subcores
    n_rows = 8 * _R
    per_w = n_rows // nw
    half = per_w // 2
    mesh = plsc.VectorSubcoreMesh(core_axis_name="c", subcore_axis_name="s")

    @functools.partial(
        pl.kernel,
        out_type=(
            jax.ShapeDtypeStruct((n_rows, _D), jnp.float32),
            jax.ShapeDtypeStruct((n_rows, _D), jnp.float32),
        ),
        mesh=mesh,
        scratch_types=[
            pltpu.VMEM((per_w,), jnp.int32),
            pltpu.VMEM((per_w,), jnp.int32),
            pltpu.VMEM((half,), jnp.int32),
            pltpu.VMEM((per_w, _D), jnp.float32),
            pltpu.VMEM((half, _SS, _D), jnp.float32),
            pltpu.SemaphoreType.DMA,
        ],
    )
    def k(t_hbm, s_hbm, ptab_hbm, ot_hbm, os_hbm,
          idx_v, idx2_v, idxh_v, rows_v, slabs_v, sem):
        wid = lax.axis_index("s") * info.num_cores + lax.axis_index("c")
        g = wid // 4
        part = wid % 4
        is_ref = g < 2
        b = jnp.where(is_ref, g, (g - 2) % 2)
        i = (g - 2) // 2
        f_t = jnp.where(is_ref, 0, 2 + 2 * i)
        f_s = jnp.where(is_ref, 0, 1 + i)
        psel = jnp.where(is_ref, 0, 1)
        base = wid * per_w
        pltpu.sync_copy(ptab_hbm.at[psel, pl.ds(part * per_w, per_w)], idx_v)
        for c in range(per_w // 16):
            sl = pl.ds(c * 16, 16)
            idx2_v[sl] = idx_v[sl] * _ST + f_t
        pltpu.async_copy(t_hbm.at[b].at[idx2_v], rows_v, sem).wait()
        pltpu.sync_copy(rows_v, ot_hbm.at[pl.ds(base, per_w)])
        for r in range(2):
            idxh_v[...] = idx_v[pl.ds(r * half, half)]
            pltpu.async_copy(s_hbm.at[b].at[idxh_v], slabs_v, sem).wait()
            pltpu.sync_copy(slabs_v.at[:, f_s],
                            os_hbm.at[pl.ds(base + r * half, half)])

    return k(tt3, st4, perm_tab)


def _simtopk_body(t_ref, r_ref, out_i_ref, rtn_scr, scr_v, scr_i):
    b = pl.program_id(0)
    c = pl.program_id(1)

    @pl.when(c == 0)
    def _init():
        scr_v[...] = jnp.full((_R, _NCAND), -jnp.inf, jnp.float32)
        scr_i[...] = jnp.zeros((_R, _NCAND), jnp.int32)
        r = r_ref[0]
        rn = jnp.maximum(jnp.sqrt(jnp.sum(r * r, axis=-1, keepdims=True)),
                         1e-12)
        rtn_scr[...] = r / rn

    nl = _PC * _ST
    fa = t_ref[0].reshape(nl, _D)
    ones = jnp.ones((1, _D), jnp.float32)
    nsq = lax.dot_general(ones, fa * fa, (((1,), (1,)), ((), ())),
                          preferred_element_type=jnp.float32)
    rec = 1.0 / jnp.maximum(jnp.sqrt(nsq), 1e-12)
    raw = lax.dot_general(rtn_scr[...], fa, (((1,), (1,)), ((), ())),
                          preferred_element_type=jnp.float32)
    sim = raw * rec

    iota = lax.broadcasted_iota(jnp.int32, (_R, nl), 1)
    valid = ((iota & 1) == 1) & (iota < (_P - c * _PC) * _ST)
    sim = jnp.where(valid, sim, -jnp.inf)

    lane = lax.broadcasted_iota(jnp.int32, (_R, _NCAND), 1)
    base = c * (_PC * _ST)
    sv = scr_v[...]
    si = scr_i[...]
    for j in range(_K):
        m = jnp.max(sim, axis=1, keepdims=True)
        pos = jnp.min(jnp.where(sim == m, iota, jnp.int32(2 ** 30)),
                      axis=1, keepdims=True)
        sim = jnp.where(iota == pos, -jnp.inf, sim)
        slot = c * _K + j
        sv = jnp.where(lane == slot, m, sv)
        si = jnp.where(lane == slot, pos + base, si)
    scr_v[...] = sv
    scr_i[...] = si

    @pl.when(c == _NC - 1)
    def _emit():
        v = scr_v[...]
        ci = scr_i[...]
        lane4 = lax.broadcasted_iota(jnp.int32, (_R, _K), 1)
        res = jnp.zeros((_R, _K), jnp.int32)
        for j in range(_K):
            m = jnp.max(v, axis=1, keepdims=True)
            pos = jnp.min(jnp.where(v == m, lane, jnp.int32(2 ** 30)),
                          axis=1, keepdims=True)
            sel = jnp.sum(jnp.where(lane == pos, ci, 0), axis=1, keepdims=True)
            res = jnp.where(lane4 == j, sel, res)
            v = jnp.where(lane == pos, -jnp.inf, v)
        out_i_ref[0] = res


def _tc_sim_topk(tt4, ref_t):
    return pl.pallas_call(
        _simtopk_body,
        grid=(_B, _NC),
        in_specs=[
            pl.BlockSpec((1, _PC, _ST, _D), lambda b, c: (b, c, 0, 0)),
            pl.BlockSpec((1, _R, _D), lambda b, c: (b, 0, 0)),
        ],
        out_specs=pl.BlockSpec((1, _R, _K), lambda b, c: (b, 0, 0)),
        out_shape=jax.ShapeDtypeStruct((_B, _R, _K), jnp.int32),
        scratch_shapes=[
            pltpu.VMEM((_R, _D), jnp.float32),
            pltpu.VMEM((_R, _NCAND), jnp.float32),
            pltpu.VMEM((_R, _NCAND), jnp.int32),
        ],
        compiler_params=pltpu.CompilerParams(
            dimension_semantics=("arbitrary", "arbitrary")),
    )(tt4, ref_t)


def _sc_topk_gather(gidx, tt3):
    info = plsc.get_sparse_core_info()
    nw = info.num_cores * info.num_subcores
    n_rows = gidx.shape[0]
    per_w = n_rows // nw
    w_per_b = nw // _B
    mesh = plsc.VectorSubcoreMesh(core_axis_name="c", subcore_axis_name="s")

    @functools.partial(
        pl.kernel,
        out_type=jax.ShapeDtypeStruct((n_rows, _D), jnp.float32),
        mesh=mesh,
        scratch_types=[
            pltpu.VMEM((per_w,), jnp.int32),
            pltpu.VMEM((per_w, _D), jnp.float32),
            pltpu.SemaphoreType.DMA,
        ],
    )
    def k(i_hbm, t_hbm, out_hbm, idx_v, rows_v, sem):
        wid = lax.axis_index("s") * info.num_cores + lax.axis_index("c")
        base = wid * per_w
        b = wid // w_per_b
        pltpu.sync_copy(i_hbm.at[pl.ds(base, per_w)], idx_v)
        pltpu.async_copy(t_hbm.at[b].at[idx_v], rows_v, sem).wait()
        pltpu.sync_copy(rows_v, out_hbm.at[pl.ds(base, per_w)])

    return k(gidx, tt3)


def _huber(pred, target):
    e = pred - target
    ae = jnp.abs(e)
    return jnp.where(ae <= 1.0, 0.5 * e * e, ae - 0.5)


def _angles_body(rt_ref, rs_ref, sht_ref, shs_ref, h_ref, out_ref):
    cd = (((1,), (1,)), ((), ()))
    row = lax.broadcasted_iota(jnp.int32, (_RK, _R), 0)
    col = lax.broadcasted_iota(jnp.int32, (_RK, _R), 1)
    repmask = (row // _K == col)
    E = repmask.astype(jnp.float32)
    ones = jnp.ones((1, _D), jnp.float32)

    def _mm(a, bm):
        return lax.dot_general(a, bm, cd, preferred_element_type=jnp.float32)

    def _rep(x):
        return lax.dot_general(E, x, (((1,), (0,)), ((), ())),
                               preferred_element_type=jnp.float32)

    acc = jnp.float32(0.0)
    for b in range(_B):
        H = h_ref[b]
        hh = jnp.sum(H * H, axis=-1, keepdims=True)
        side = []
        for r_ref_ in (rt_ref, rs_ref):
            ref = r_ref_[b]
            rhm = _mm(H, ref)
            rh = jnp.sum(jnp.where(repmask, rhm, 0.0),
                         axis=1, keepdims=True)
            rr = _rep(jnp.sum(ref * ref, axis=-1, keepdims=True))
            side.append((ref, rh, rr))
        for i in range(len(_SHT)):
            angles = []
            for (sh_ref_, (ref, rh, rr)) in ((sht_ref, side[0]),
                                             (shs_ref, side[1])):
                sh = sh_ref_[i, b]
                rs2 = _rep(_mm(ref, sh))
                sh2 = _mm(H, sh)
                ss = _mm(ones, sh * sh)
                dot1 = sh2 - rs2 - rh + rr
                na1 = jnp.maximum(jnp.sqrt(jnp.maximum(ss + rr - 2.0 * rs2, 0.0)), _EPS)
                nb1 = jnp.maximum(jnp.sqrt(jnp.maximum(hh + rr - 2.0 * rh, 0.0)), _EPS)
                a1 = dot1 / (na1 * nb1)
                dot2 = rs2 - sh2 - rh + hh
                na2 = jnp.maximum(jnp.sqrt(jnp.maximum(rr + hh - 2.0 * rh, 0.0)), _EPS)
                nb2 = jnp.maximum(jnp.sqrt(jnp.maximum(ss + hh - 2.0 * sh2, 1e-12)), _EPS)
                a2 = dot2 / (na2 * nb2)
                dot3 = rh - sh2 - rs2 + ss
                na3 = jnp.maximum(jnp.sqrt(jnp.maximum(rr + ss - 2.0 * rs2, 0.0)), _EPS)
                nb3 = jnp.maximum(jnp.sqrt(jnp.maximum(hh + ss - 2.0 * sh2, 1e-12)), _EPS)
                a3 = dot3 / (na3 * nb3)
                angles.append((a1, a2, a3))
            (t1, t2, t3), (s1, s2, s3) = angles
            acc = acc + jnp.sum(_huber(s1, t1)) + jnp.sum(_huber(s2, t2)) \
                      + jnp.sum(_huber(s3, t3))
    total = float(len(_SHT) * _B * _R * _S * _K)
    out_ref[...] = jnp.reshape(acc / total, (1, 1))


def _tc_angles(ref_t, ref_s, shared_t, shared_s, h):
    return pl.pallas_call(
        _angles_body,
        out_shape=jax.ShapeDtypeStruct((1, 1), jnp.float32),
    )(ref_t, ref_s, shared_t, shared_s, h)


def kernel(teacher_feats, student_feats, ref_perm, shared_perm):
    perm_tab = jnp.stack([ref_perm, shared_perm])

    tt4 = jnp.transpose(teacher_feats, (0, 2, 1, 3))
    tt3 = tt4.reshape(_B, _P * _ST, _D)
    st4 = jnp.transpose(student_feats, (0, 2, 1, 3))

    out_t, out_s = _sc_perm_gather(tt3, st4, perm_tab)
    ref_t = out_t[:_B * _R].reshape(_B, _R, _D)
    shared_t = out_t[_B * _R:].reshape(len(_SHT), _B, _S, _D)
    ref_s = out_s[:_B * _R].reshape(_B, _R, _D)
    shared_s = out_s[_B * _R:].reshape(len(_SHS), _B, _S, _D)

    gidx = _tc_sim_topk(tt4, ref_t)
    h = _sc_topk_gather(gidx.reshape(_B * _R * _K), tt3).reshape(_B, _RK, _D)

    out = _tc_angles(ref_t, ref_s, shared_t, shared_s, h)
    return out[0, 0]


# speedup vs baseline: 1.1074x; 1.1074x over previous
import functools

import jax
import jax.numpy as jnp
from jax import lax
from jax.experimental import pallas as pl
from jax.experimental.pallas import tpu as pltpu
from jax.experimental.pallas import tpu_sc as plsc

_B, _ST, _SS, _P, _D = 2, 8, 4, 1369, 1024
_R = 128
_S = 128
_K = 4
_SHT = (2, 4, 6)
_SHS = (1, 2, 3)
_EPS = 1e-8
_RK = _R * _K
_PC = 344
_NC = 4
_NCAND = _NC * _K


def _sc_perm_gather_t(tt3, perm_tab):
    info = plsc.get_sparse_core_info()
    nw = info.num_cores * info.num_subcores
    n_rows = 8 * _R
    per_w = n_rows // nw
    mesh = plsc.VectorSubcoreMesh(core_axis_name="c", subcore_axis_name="s")

    @functools.partial(
        pl.kernel,
        out_type=jax.ShapeDtypeStruct((n_rows, _D), jnp.float32),
        mesh=mesh,
        scratch_types=[
            pltpu.VMEM((per_w,), jnp.int32),
            pltpu.VMEM((per_w,), jnp.int32),
            pltpu.VMEM((per_w, _D), jnp.float32),
            pltpu.SemaphoreType.DMA,
        ],
    )
    def k(t_hbm, ptab_hbm, ot_hbm, idx_v, idx2_v, rows_v, sem):
        wid = lax.axis_index("s") * info.num_cores + lax.axis_index("c")
        g = wid // 4
        part = wid % 4
        is_ref = g < 2
        b = jnp.where(is_ref, g, (g - 2) % 2)
        i = (g - 2) // 2
        f_t = jnp.where(is_ref, 0, 2 + 2 * i)
        psel = jnp.where(is_ref, 0, 1)
        base = wid * per_w
        pltpu.sync_copy(ptab_hbm.at[psel, pl.ds(part * per_w, per_w)], idx_v)
        for c in range(per_w // 16):
            sl = pl.ds(c * 16, 16)
            idx2_v[sl] = idx_v[sl] * _ST + f_t
        pltpu.async_copy(t_hbm.at[b].at[idx2_v], rows_v, sem).wait()
        pltpu.sync_copy(rows_v, ot_hbm.at[pl.ds(base, per_w)])

    return k(tt3, perm_tab)


def _sc_perm_gather_s(st4, perm_tab):
    info = plsc.get_sparse_core_info()
    nw = info.num_cores * info.num_subcores
    n_rows = 8 * _R
    per_w = n_rows // nw
    half = per_w // 2
    mesh = plsc.VectorSubcoreMesh(core_axis_name="c", subcore_axis_name="s")

    @functools.partial(
        pl.kernel,
        out_type=jax.ShapeDtypeStruct((n_rows, _D), jnp.float32),
        mesh=mesh,
        scratch_types=[
            pltpu.VMEM((per_w,), jnp.int32),
            pltpu.VMEM((half,), jnp.int32),
            pltpu.VMEM((half, _SS, _D), jnp.float32),
            pltpu.SemaphoreType.DMA,
        ],
    )
    def k(s_hbm, ptab_hbm, os_hbm, idx_v, idxh_v, slabs_v, sem):
        wid = lax.axis_index("s") * info.num_cores + lax.axis_index("c")
        g = wid // 4
        part = wid % 4
        is_ref = g < 2
        b = jnp.where(is_ref, g, (g - 2) % 2)
        i = (g - 2) // 2
        f_s = jnp.where(is_ref, 0, 1 + i)
        psel = jnp.where(is_ref, 0, 1)
        base = wid * per_w
        pltpu.sync_copy(ptab_hbm.at[psel, pl.ds(part * per_w, per_w)], idx_v)
        for r in range(2):
            idxh_v[...] = idx_v[pl.ds(r * half, half)]
            pltpu.async_copy(s_hbm.at[b].at[idxh_v], slabs_v, sem).wait()
            pltpu.sync_copy(slabs_v.at[:, f_s],
                            os_hbm.at[pl.ds(base + r * half, half)])

    return k(st4, perm_tab)


def _simtopk_body(t_ref, r_ref, out_i_ref, scr_v, scr_i):
    b = pl.program_id(0)
    c = pl.program_id(1)

    @pl.when(c == 0)
    def _init():
        scr_v[...] = jnp.full((_R, _NCAND), -jnp.inf, jnp.float32)
        scr_i[...] = jnp.zeros((_R, _NCAND), jnp.int32)

    nl = _PC * _ST
    fa = t_ref[0].reshape(nl, _D)
    fn = jnp.maximum(jnp.sqrt(jnp.sum(fa * fa, axis=-1, keepdims=True)),
                     1e-12)
    rec_row = lax.transpose(1.0 / fn, (1, 0))
    raw = lax.dot_general(r_ref[0], fa, (((1,), (1,)), ((), ())),
                          preferred_element_type=jnp.float32)
    sim = raw * rec_row

    iota = lax.broadcasted_iota(jnp.int32, (_R, nl), 1)
    valid = ((iota & 1) == 1) & (iota < (_P - c * _PC) * _ST)
    sim = jnp.where(valid, sim, -jnp.inf)

    lane = lax.broadcasted_iota(jnp.int32, (_R, _NCAND), 1)
    base = c * (_PC * _ST)
    sv = scr_v[...]
    si = scr_i[...]
    for j in range(_K):
        m = jnp.max(sim, axis=1, keepdims=True)
        pos = jnp.min(jnp.where(sim == m, iota, jnp.int32(2 ** 30)),
                      axis=1, keepdims=True)
        sim = jnp.where(iota == pos, -jnp.inf, sim)
        slot = c * _K + j
        sv = jnp.where(lane == slot, m, sv)
        si = jnp.where(lane == slot, pos + base, si)
    scr_v[...] = sv
    scr_i[...] = si

    @pl.when(c == _NC - 1)
    def _emit():
        v = scr_v[...]
        ci = scr_i[...]
        lane4 = lax.broadcasted_iota(jnp.int32, (_R, _K), 1)
        res = jnp.zeros((_R, _K), jnp.int32)
        for j in range(_K):
            m = jnp.max(v, axis=1, keepdims=True)
            pos = jnp.min(jnp.where(v == m, lane, jnp.int32(2 ** 30)),
                          axis=1, keepdims=True)
            sel = jnp.sum(jnp.where(lane == pos, ci, 0), axis=1, keepdims=True)
            res = jnp.where(lane4 == j, sel, res)
            v = jnp.where(lane == pos, -jnp.inf, v)
        out_i_ref[0] = res


def _tc_sim_topk(tt4, ref_t):
    return pl.pallas_call(
        _simtopk_body,
        grid=(_B, _NC),
        in_specs=[
            pl.BlockSpec((1, _PC, _ST, _D), lambda b, c: (b, c, 0, 0)),
            pl.BlockSpec((1, _R, _D), lambda b, c: (b, 0, 0)),
        ],
        out_specs=pl.BlockSpec((1, _R, _K), lambda b, c: (b, 0, 0)),
        out_shape=jax.ShapeDtypeStruct((_B, _R, _K), jnp.int32),
        scratch_shapes=[
            pltpu.VMEM((_R, _NCAND), jnp.float32),
            pltpu.VMEM((_R, _NCAND), jnp.int32),
        ],
        compiler_params=pltpu.CompilerParams(
            dimension_semantics=("arbitrary", "arbitrary")),
    )(tt4, ref_t)


def _sc_topk_gather(gidx, tt3):
    info = plsc.get_sparse_core_info()
    nw = info.num_cores * info.num_subcores
    n_rows = gidx.shape[0]
    per_w = n_rows // nw
    w_per_b = nw // _B
    mesh = plsc.VectorSubcoreMesh(core_axis_name="c", subcore_axis_name="s")

    @functools.partial(
        pl.kernel,
        out_type=jax.ShapeDtypeStruct((n_rows, _D), jnp.float32),
        mesh=mesh,
        scratch_types=[
            pltpu.VMEM((per_w,), jnp.int32),
            pltpu.VMEM((per_w, _D), jnp.float32),
            pltpu.SemaphoreType.DMA,
        ],
    )
    def k(i_hbm, t_hbm, out_hbm, idx_v, rows_v, sem):
        wid = lax.axis_index("s") * info.num_cores + lax.axis_index("c")
        base = wid * per_w
        b = wid // w_per_b
        pltpu.sync_copy(i_hbm.at[pl.ds(base, per_w)], idx_v)
        pltpu.async_copy(t_hbm.at[b].at[idx_v], rows_v, sem).wait()
        pltpu.sync_copy(rows_v, out_hbm.at[pl.ds(base, per_w)])

    return k(gidx, tt3)


def _huber(pred, target):
    e = pred - target
    ae = jnp.abs(e)
    return jnp.where(ae <= 1.0, 0.5 * e * e, ae - 0.5)


def _angles_body(rt_ref, rs_ref, sht_ref, shs_ref, h_ref, out_ref):
    cd = (((1,), (1,)), ((), ()))
    row = lax.broadcasted_iota(jnp.int32, (_RK, _R), 0)
    col = lax.broadcasted_iota(jnp.int32, (_RK, _R), 1)
    repmask = (row // _K == col)
    E = repmask.astype(jnp.float32)
    ones = jnp.ones((1, _D), jnp.float32)

    def _mm(a, bm):
        return lax.dot_general(a, bm, cd, preferred_element_type=jnp.float32)

    def _rep(x):
        return lax.dot_general(E, x, (((1,), (0,)), ((), ())),
                               preferred_element_type=jnp.float32)

    acc = jnp.float32(0.0)
    for b in range(_B):
        H = h_ref[b]
        hh = jnp.sum(H * H, axis=-1, keepdims=True)
        side = []
        for r_ref_ in (rt_ref, rs_ref):
            ref = r_ref_[b]
            rhm = _mm(H, ref)
            rh = jnp.sum(jnp.where(repmask, rhm, 0.0),
                         axis=1, keepdims=True)
            rr = _rep(jnp.sum(ref * ref, axis=-1, keepdims=True))
            side.append((ref, rh, rr))
        for i in range(len(_SHT)):
            angles = []
            for (sh_ref_, (ref, rh, rr)) in ((sht_ref, side[0]),
                                             (shs_ref, side[1])):
                sh = sh_ref_[i, b]
                rs2 = _rep(_mm(ref, sh))
                sh2 = _mm(H, sh)
                ss = _mm(ones, sh * sh)
                dot1 = sh2 - rs2 - rh + rr
                na1 = jnp.maximum(jnp.sqrt(jnp.maximum(ss + rr - 2.0 * rs2, 0.0)), _EPS)
                nb1 = jnp.maximum(jnp.sqrt(jnp.maximum(hh + rr - 2.0 * rh, 0.0)), _EPS)
                a1 = dot1 / (na1 * nb1)
                dot2 = rs2 - sh2 - rh + hh
                na2 = jnp.maximum(jnp.sqrt(jnp.maximum(rr + hh - 2.0 * rh, 0.0)), _EPS)
                nb2 = jnp.maximum(jnp.sqrt(jnp.maximum(ss + hh - 2.0 * sh2, 1e-12)), _EPS)
                a2 = dot2 / (na2 * nb2)
                dot3 = rh - sh2 - rs2 + ss
                na3 = jnp.maximum(jnp.sqrt(jnp.maximum(rr + ss - 2.0 * rs2, 0.0)), _EPS)
                nb3 = jnp.maximum(jnp.sqrt(jnp.maximum(hh + ss - 2.0 * sh2, 1e-12)), _EPS)
                a3 = dot3 / (na3 * nb3)
                angles.append((a1, a2, a3))
            (t1, t2, t3), (s1, s2, s3) = angles
            acc = acc + jnp.sum(_huber(s1, t1)) + jnp.sum(_huber(s2, t2)) \
                      + jnp.sum(_huber(s3, t3))
    total = float(len(_SHT) * _B * _R * _S * _K)
    out_ref[...] = jnp.reshape(acc / total, (1, 1))


def _tc_angles(ref_t, ref_s, shared_t, shared_s, h):
    return pl.pallas_call(
        _angles_body,
        out_shape=jax.ShapeDtypeStruct((1, 1), jnp.float32),
    )(ref_t, ref_s, shared_t, shared_s, h)


def kernel(teacher_feats, student_feats, ref_perm, shared_perm):
    perm_tab = jnp.stack([ref_perm, shared_perm])

    tt4 = jnp.transpose(teacher_feats, (0, 2, 1, 3))
    tt3 = tt4.reshape(_B, _P * _ST, _D)
    st4 = jnp.transpose(student_feats, (0, 2, 1, 3))

    out_t = _sc_perm_gather_t(tt3, perm_tab)
    out_s = _sc_perm_gather_s(st4, perm_tab)
    ref_t = out_t[:_B * _R].reshape(_B, _R, _D)
    shared_t = out_t[_B * _R:].reshape(len(_SHT), _B, _S, _D)
    ref_s = out_s[:_B * _R].reshape(_B, _R, _D)
    shared_s = out_s[_B * _R:].reshape(len(_SHS), _B, _S, _D)

    gidx = _tc_sim_topk(tt4, ref_t)
    h = _sc_topk_gather(gidx.reshape(_B * _R * _K), tt3).reshape(_B, _RK, _D)

    out = _tc_angles(ref_t, ref_s, shared_t, shared_s, h)
    return out[0, 0]


# speedup vs baseline: 1.2238x; 1.1051x over previous
import functools

import jax
import jax.numpy as jnp
from jax import lax
from jax.experimental import pallas as pl
from jax.experimental.pallas import tpu as pltpu
from jax.experimental.pallas import tpu_sc as plsc

_B, _ST, _SS, _P, _D = 2, 8, 4, 1369, 1024
_R = 128
_S = 128
_K = 4
_SHT = (2, 4, 6)
_SHS = (1, 2, 3)
_EPS = 1e-8
_RK = _R * _K
_PC = 344
_NC = 4
_NCAND = _NC * _K


def _sc_perm_gather_t(tt3, perm_tab):
    info = plsc.get_sparse_core_info()
    nw = info.num_cores * info.num_subcores
    n_rows = 8 * _R
    per_w = n_rows // nw
    mesh = plsc.VectorSubcoreMesh(core_axis_name="c", subcore_axis_name="s")

    @functools.partial(
        pl.kernel,
        out_type=(
            jax.ShapeDtypeStruct((_B, _R, _D), jnp.float32),
            jax.ShapeDtypeStruct((len(_SHT), _B, _S, _D), jnp.float32),
        ),
        mesh=mesh,
        scratch_types=[
            pltpu.VMEM((per_w,), jnp.int32),
            pltpu.VMEM((per_w,), jnp.int32),
            pltpu.VMEM((per_w, _D), jnp.float32),
            pltpu.SemaphoreType.DMA,
        ],
    )
    def k(t_hbm, ptab_hbm, oref_hbm, osh_hbm, idx_v, idx2_v, rows_v, sem):
        wid = lax.axis_index("s") * info.num_cores + lax.axis_index("c")
        g = wid // 4
        part = wid % 4
        is_ref = g < 2
        b = jnp.where(is_ref, g, (g - 2) % 2)
        i = (g - 2) // 2
        f_t = jnp.where(is_ref, 0, 2 + 2 * i)
        psel = jnp.where(is_ref, 0, 1)
        pltpu.sync_copy(ptab_hbm.at[psel, pl.ds(part * per_w, per_w)], idx_v)
        for c in range(per_w // 16):
            sl = pl.ds(c * 16, 16)
            idx2_v[sl] = idx_v[sl] * _ST + f_t
        pltpu.async_copy(t_hbm.at[b].at[idx2_v], rows_v, sem).wait()
        row0 = part * per_w

        @pl.when(is_ref)
        def _():
            pltpu.sync_copy(rows_v, oref_hbm.at[b, pl.ds(row0, per_w)])

        @pl.when(jnp.logical_not(is_ref))
        def _():
            pltpu.sync_copy(rows_v, osh_hbm.at[i, b, pl.ds(row0, per_w)])

    return k(tt3, perm_tab)


def _sc_perm_gather_s(st4, perm_tab):
    info = plsc.get_sparse_core_info()
    nw = info.num_cores * info.num_subcores
    n_rows = 8 * _R
    per_w = n_rows // nw
    half = per_w // 2
    mesh = plsc.VectorSubcoreMesh(core_axis_name="c", subcore_axis_name="s")

    @functools.partial(
        pl.kernel,
        out_type=(
            jax.ShapeDtypeStruct((_B, _R, _D), jnp.float32),
            jax.ShapeDtypeStruct((len(_SHS), _B, _S, _D), jnp.float32),
        ),
        mesh=mesh,
        scratch_types=[
            pltpu.VMEM((per_w,), jnp.int32),
            pltpu.VMEM((half,), jnp.int32),
            pltpu.VMEM((half, _SS, _D), jnp.float32),
            pltpu.SemaphoreType.DMA,
        ],
    )
    def k(s_hbm, ptab_hbm, oref_hbm, osh_hbm, idx_v, idxh_v, slabs_v, sem):
        wid = lax.axis_index("s") * info.num_cores + lax.axis_index("c")
        g = wid // 4
        part = wid % 4
        is_ref = g < 2
        b = jnp.where(is_ref, g, (g - 2) % 2)
        i = (g - 2) // 2
        f_s = jnp.where(is_ref, 0, 1 + i)
        psel = jnp.where(is_ref, 0, 1)
        pltpu.sync_copy(ptab_hbm.at[psel, pl.ds(part * per_w, per_w)], idx_v)
        for r in range(2):
            idxh_v[...] = idx_v[pl.ds(r * half, half)]
            pltpu.async_copy(s_hbm.at[b].at[idxh_v], slabs_v, sem).wait()
            row0 = part * per_w + r * half

            @pl.when(is_ref)
            def _():
                pltpu.sync_copy(slabs_v.at[:, f_s],
                                oref_hbm.at[b, pl.ds(row0, half)])

            @pl.when(jnp.logical_not(is_ref))
            def _():
                pltpu.sync_copy(slabs_v.at[:, f_s],
                                osh_hbm.at[i, b, pl.ds(row0, half)])

    return k(st4, perm_tab)


def _simtopk_body(t_ref, r_ref, out_i_ref, scr_v, scr_i):
    b = pl.program_id(0)
    c = pl.program_id(1)

    @pl.when(c == 0)
    def _init():
        scr_v[...] = jnp.full((_R, _NCAND), -jnp.inf, jnp.float32)
        scr_i[...] = jnp.zeros((_R, _NCAND), jnp.int32)

    nl = _PC * _ST
    fa = t_ref[0].reshape(nl, _D)
    fn = jnp.maximum(jnp.sqrt(jnp.sum(fa * fa, axis=-1, keepdims=True)),
                     1e-12)
    rec_row = lax.transpose(1.0 / fn, (1, 0))
    raw = lax.dot_general(r_ref[0], fa, (((1,), (1,)), ((), ())),
                          preferred_element_type=jnp.float32)
    sim = raw * rec_row

    iota = lax.broadcasted_iota(jnp.int32, (_R, nl), 1)
    valid = ((iota & 1) == 1) & (iota < (_P - c * _PC) * _ST)
    sim = jnp.where(valid, sim, -jnp.inf)

    lane = lax.broadcasted_iota(jnp.int32, (_R, _NCAND), 1)
    base = c * (_PC * _ST)
    sv = scr_v[...]
    si = scr_i[...]
    for j in range(_K):
        m = jnp.max(sim, axis=1, keepdims=True)
        pos = jnp.min(jnp.where(sim == m, iota, jnp.int32(2 ** 30)),
                      axis=1, keepdims=True)
        sim = jnp.where(iota == pos, -jnp.inf, sim)
        slot = c * _K + j
        sv = jnp.where(lane == slot, m, sv)
        si = jnp.where(lane == slot, pos + base, si)
    scr_v[...] = sv
    scr_i[...] = si

    @pl.when(c == _NC - 1)
    def _emit():
        v = scr_v[...]
        ci = scr_i[...]
        lane4 = lax.broadcasted_iota(jnp.int32, (_R, _K), 1)
        res = jnp.zeros((_R, _K), jnp.int32)
        for j in range(_K):
            m = jnp.max(v, axis=1, keepdims=True)
            pos = jnp.min(jnp.where(v == m, lane, jnp.int32(2 ** 30)),
                          axis=1, keepdims=True)
            sel = jnp.sum(jnp.where(lane == pos, ci, 0), axis=1, keepdims=True)
            res = jnp.where(lane4 == j, sel, res)
            v = jnp.where(lane == pos, -jnp.inf, v)
        out_i_ref[0] = res


def _tc_sim_topk(tt4, ref_t):
    return pl.pallas_call(
        _simtopk_body,
        grid=(_B, _NC),
        in_specs=[
            pl.BlockSpec((1, _PC, _ST, _D), lambda b, c: (b, c, 0, 0)),
            pl.BlockSpec((1, _R, _D), lambda b, c: (b, 0, 0)),
        ],
        out_specs=pl.BlockSpec((1, _R, _K), lambda b, c: (b, 0, 0)),
        out_shape=jax.ShapeDtypeStruct((_B, _R, _K), jnp.int32),
        scratch_shapes=[
            pltpu.VMEM((_R, _NCAND), jnp.float32),
            pltpu.VMEM((_R, _NCAND), jnp.int32),
        ],
        compiler_params=pltpu.CompilerParams(
            dimension_semantics=("arbitrary", "arbitrary")),
    )(tt4, ref_t)


def _sc_topk_gather(gidx, tt3):
    info = plsc.get_sparse_core_info()
    nw = info.num_cores * info.num_subcores
    n_rows = gidx.shape[0]
    per_w = n_rows // nw
    w_per_b = nw // _B
    mesh = plsc.VectorSubcoreMesh(core_axis_name="c", subcore_axis_name="s")

    @functools.partial(
        pl.kernel,
        out_type=jax.ShapeDtypeStruct((n_rows, _D), jnp.float32),
        mesh=mesh,
        scratch_types=[
            pltpu.VMEM((per_w,), jnp.int32),
            pltpu.VMEM((per_w, _D), jnp.float32),
            pltpu.SemaphoreType.DMA,
        ],
    )
    def k(i_hbm, t_hbm, out_hbm, idx_v, rows_v, sem):
        wid = lax.axis_index("s") * info.num_cores + lax.axis_index("c")
        base = wid * per_w
        b = wid // w_per_b
        pltpu.sync_copy(i_hbm.at[pl.ds(base, per_w)], idx_v)
        pltpu.async_copy(t_hbm.at[b].at[idx_v], rows_v, sem).wait()
        pltpu.sync_copy(rows_v, out_hbm.at[pl.ds(base, per_w)])

    return k(gidx, tt3)


def _huber(pred, target):
    e = pred - target
    ae = jnp.abs(e)
    return jnp.where(ae <= 1.0, 0.5 * e * e, ae - 0.5)


def _angles_body(rt_ref, rs_ref, sht_ref, shs_ref, h_ref, out_ref):
    cd = (((1,), (1,)), ((), ()))
    row = lax.broadcasted_iota(jnp.int32, (_RK, _R), 0)
    col = lax.broadcasted_iota(jnp.int32, (_RK, _R), 1)
    repmask = (row // _K == col)
    E = repmask.astype(jnp.float32)
    ones = jnp.ones((1, _D), jnp.float32)

    def _mm(a, bm):
        return lax.dot_general(a, bm, cd, preferred_element_type=jnp.float32)

    def _rep(x):
        return lax.dot_general(E, x, (((1,), (0,)), ((), ())),
                               preferred_element_type=jnp.float32)

    b = pl.program_id(0)
    acc = jnp.float32(0.0)
    if True:
        H = h_ref[0]
        hh = jnp.sum(H * H, axis=-1, keepdims=True)
        side = []
        for r_ref_ in (rt_ref, rs_ref):
            ref = r_ref_[0]
            rhm = _mm(H, ref)
            rh = jnp.sum(jnp.where(repmask, rhm, 0.0),
                         axis=1, keepdims=True)
            rr = _rep(jnp.sum(ref * ref, axis=-1, keepdims=True))
            side.append((ref, rh, rr))
        for i in range(len(_SHT)):
            angles = []
            for (sh_ref_, (ref, rh, rr)) in ((sht_ref, side[0]),
                                             (shs_ref, side[1])):
                sh = sh_ref_[i, 0]
                rs2 = _rep(_mm(ref, sh))
                sh2 = _mm(H, sh)
                ss = _mm(ones, sh * sh)
                dot1 = sh2 - rs2 - rh + rr
                na1 = jnp.maximum(jnp.sqrt(jnp.maximum(ss + rr - 2.0 * rs2, 0.0)), _EPS)
                nb1 = jnp.maximum(jnp.sqrt(jnp.maximum(hh + rr - 2.0 * rh, 0.0)), _EPS)
                a1 = dot1 / (na1 * nb1)
                dot2 = rs2 - sh2 - rh + hh
                na2 = jnp.maximum(jnp.sqrt(jnp.maximum(rr + hh - 2.0 * rh, 0.0)), _EPS)
                nb2 = jnp.maximum(jnp.sqrt(jnp.maximum(ss + hh - 2.0 * sh2, 1e-12)), _EPS)
                a2 = dot2 / (na2 * nb2)
                dot3 = rh - sh2 - rs2 + ss
                na3 = jnp.maximum(jnp.sqrt(jnp.maximum(rr + ss - 2.0 * rs2, 0.0)), _EPS)
                nb3 = jnp.maximum(jnp.sqrt(jnp.maximum(hh + ss - 2.0 * sh2, 1e-12)), _EPS)
                a3 = dot3 / (na3 * nb3)
                angles.append((a1, a2, a3))
            (t1, t2, t3), (s1, s2, s3) = angles
            acc = acc + jnp.sum(_huber(s1, t1)) + jnp.sum(_huber(s2, t2)) \
                      + jnp.sum(_huber(s3, t3))
    total = float(len(_SHT) * _B * _R * _S * _K)
    part = jnp.reshape(acc / total, (1, 1))

    @pl.when(b == 0)
    def _():
        out_ref[...] = part

    @pl.when(b != 0)
    def _():
        out_ref[...] = out_ref[...] + part


def _tc_angles(ref_t, ref_s, shared_t, shared_s, h):
    return pl.pallas_call(
        _angles_body,
        grid=(_B,),
        in_specs=[
            pl.BlockSpec((1, _R, _D), lambda b: (b, 0, 0)),
            pl.BlockSpec((1, _R, _D), lambda b: (b, 0, 0)),
            pl.BlockSpec((len(_SHT), 1, _S, _D), lambda b: (0, b, 0, 0)),
            pl.BlockSpec((len(_SHS), 1, _S, _D), lambda b: (0, b, 0, 0)),
            pl.BlockSpec((1, _RK, _D), lambda b: (b, 0, 0)),
        ],
        out_specs=pl.BlockSpec((1, 1), lambda b: (0, 0)),
        out_shape=jax.ShapeDtypeStruct((1, 1), jnp.float32),
        compiler_params=pltpu.CompilerParams(
            dimension_semantics=("arbitrary",)),
    )(ref_t, ref_s, shared_t, shared_s, h)


def kernel(teacher_feats, student_feats, ref_perm, shared_perm):
    perm_tab = jnp.stack([ref_perm, shared_perm])

    tt4 = jnp.transpose(teacher_feats, (0, 2, 1, 3))
    tt3 = tt4.reshape(_B, _P * _ST, _D)
    st4 = jnp.transpose(student_feats, (0, 2, 1, 3))

    ref_t, shared_t = _sc_perm_gather_t(tt3, perm_tab)
    ref_s, shared_s = _sc_perm_gather_s(st4, perm_tab)

    gidx = _tc_sim_topk(tt4, ref_t)
    h = _sc_topk_gather(gidx.reshape(_B * _R * _K), tt3).reshape(_B, _RK, _D)

    out = _tc_angles(ref_t, ref_s, shared_t, shared_s, h)
    return out[0, 0]
